# baseline (speedup 1.0000x reference)
import jax
import jax.numpy as jnp
from jax import lax
from jax.experimental import pallas as pl
from jax.experimental.pallas import tpu as pltpu
from jax.experimental.pallas import tpu_sc as plsc

_NC = 2
_NS = 16
_NW = _NC * _NS
_L = 16
_NBUF = 2
_NQ = 4


def _chunk_add(t_v, in_b, out_b, tbase, n_vregs):
    @plsc.parallel_loop(0, n_vregs, unroll=8)
    def body(i):
        toff = tbase + ((i >> 6) << 7) + ((i & 7) << 4)
        out_b[pl.ds(i * _L, _L)] = in_b[pl.ds(i * _L, _L)] + t_v[pl.ds(toff, _L)]


def _sc_body(x_hbm, t_hbm, o_hbm, t_v, in_bufs, out_bufs, ld_sems, st_sems):
    K, M = x_hbm.shape
    cw = M // _NQ
    n_vregs = cw // _L
    tq = t_hbm.shape[0] // _NQ
    kpw = K // _NW
    nch = kpw * _NQ
    ng = nch // _NBUF
    wid = lax.axis_index("s") * _NC + lax.axis_index("c")
    kbase = wid * kpw

    pltpu.sync_copy(t_hbm, t_v)

    def chunk_slice(ref, e):
        return ref.at[kbase + e // _NQ, pl.ds(cw * (e % _NQ), cw)]

    for j in range(_NBUF):
        pltpu.async_copy(chunk_slice(x_hbm, j), in_bufs[j], ld_sems[j])

    def slot(g, j, *, first, last):
        e = g * _NBUF + j
        pltpu.make_async_copy(chunk_slice(x_hbm, e), in_bufs[j], ld_sems[j]).wait()
        if not first:
            pltpu.make_async_copy(out_bufs[j], chunk_slice(o_hbm, e), st_sems[j]).wait()
        _chunk_add(t_v, in_bufs[j], out_bufs[j], tq * (e % _NQ), n_vregs)
        pltpu.async_copy(out_bufs[j], chunk_slice(o_hbm, e), st_sems[j])
        if not last:
            pltpu.async_copy(chunk_slice(x_hbm, e + _NBUF), in_bufs[j], ld_sems[j])

    for j in range(_NBUF):
        slot(0, j, first=True, last=False)

    def group(g, c):
        for j in range(_NBUF):
            slot(g, j, first=False, last=False)
        return c

    lax.fori_loop(1, ng - 1, group, 0)

    for j in range(_NBUF):
        slot(ng - 1, j, first=False, last=True)

    for j in range(_NBUF):
        e = (ng - 1) * _NBUF + j
        pltpu.make_async_copy(out_bufs[j], chunk_slice(o_hbm, e), st_sems[j]).wait()


def kernel(input_tensor, pos_table):
    B, S, E = input_tensor.shape
    D = S * E
    K = B // 8
    M = 8 * D
    cw = M // _NQ

    x = (input_tensor.reshape(K, 8, D // 128, 128)
         .transpose(0, 2, 1, 3)
         .reshape(K, M))
    t = pos_table.reshape(D)

    mesh = plsc.VectorSubcoreMesh(core_axis_name="c", subcore_axis_name="s",
                                  num_cores=_NC)
    scratch = (
        [pltpu.VMEM((D,), jnp.float32)]
        + [pltpu.VMEM((cw,), jnp.float32)] * _NBUF
        + [pltpu.VMEM((cw,), jnp.float32)] * _NBUF
        + [pltpu.SemaphoreType.DMA] * (2 * _NBUF)
    )

    def body(x_ref, t_ref, o_ref, *scr):
        t_v = scr[0]
        in_bufs = scr[1:1 + _NBUF]
        out_bufs = scr[1 + _NBUF:1 + 2 * _NBUF]
        ld_sems = scr[1 + 2 * _NBUF:1 + 3 * _NBUF]
        st_sems = scr[1 + 3 * _NBUF:1 + 4 * _NBUF]
        _sc_body(x_ref, t_ref, o_ref, t_v, in_bufs, out_bufs, ld_sems, st_sems)

    run = pl.kernel(
        body,
        out_type=jax.ShapeDtypeStruct((K, M), jnp.float32),
        mesh=mesh,
        scratch_types=scratch,
    )
    out = run(x, t)
    return (out.reshape(K, D // 128, 8, 128)
            .transpose(0, 2, 1, 3)
            .reshape(B, S, E))


# speedup vs baseline: 3.2308x; 3.2308x over previous
import jax
import jax.numpy as jnp
from jax import lax
from jax.experimental import pallas as pl
from jax.experimental.pallas import tpu as pltpu
from jax.experimental.pallas import tpu_sc as plsc

_NC = 2
_NS = 16
_NW = _NC * _NS
_L = 16
_NBUF = 2
_NQ = 4


def _chunk_add(t_v, in_b, out_b, tbase, n_vregs):
    @plsc.parallel_loop(0, n_vregs, unroll=8)
    def body(i):
        s = (i >> 3) & 7
        col = pl.multiple_of(((i >> 6) << 7) + ((i & 7) << 4), _L)
        tcol = pl.multiple_of(tbase + col, _L)
        out_b[s, pl.ds(col, _L)] = in_b[s, pl.ds(col, _L)] + t_v[pl.ds(tcol, _L)]


def _sc_body(x_hbm, t_hbm, o_hbm, t_v, in_bufs, out_bufs, ld_sems, st_sems):
    B, D = x_hbm.shape
    cw = D // _NQ
    n_vregs = 8 * cw // _L
    kpw = (B // 8) // _NW
    nch = kpw * _NQ
    ng = nch // _NBUF
    wid = lax.axis_index("s") * _NC + lax.axis_index("c")
    kbase = wid * kpw

    pltpu.sync_copy(t_hbm, t_v)

    def chunk_slice(ref, e):
        tr = kbase + e // _NQ
        return ref.at[pl.ds(tr * 8, 8), pl.ds(cw * (e % _NQ), cw)]

    for j in range(_NBUF):
        pltpu.async_copy(chunk_slice(x_hbm, j), in_bufs[j], ld_sems[j])

    def slot(g, j, *, first, last):
        e = g * _NBUF + j
        pltpu.make_async_copy(chunk_slice(x_hbm, e), in_bufs[j], ld_sems[j]).wait()
        if not first:
            pltpu.make_async_copy(out_bufs[j], chunk_slice(o_hbm, e), st_sems[j]).wait()
        _chunk_add(t_v, in_bufs[j], out_bufs[j], cw * (e % _NQ), n_vregs)
        pltpu.async_copy(out_bufs[j], chunk_slice(o_hbm, e), st_sems[j])
        if not last:
            pltpu.async_copy(chunk_slice(x_hbm, e + _NBUF), in_bufs[j], ld_sems[j])

    for j in range(_NBUF):
        slot(0, j, first=True, last=False)

    def group(g, c):
        for j in range(_NBUF):
            slot(g, j, first=False, last=False)
        return c

    lax.fori_loop(1, ng - 1, group, 0)

    for j in range(_NBUF):
        slot(ng - 1, j, first=False, last=True)

    for j in range(_NBUF):
        e = (ng - 1) * _NBUF + j
        pltpu.make_async_copy(out_bufs[j], chunk_slice(o_hbm, e), st_sems[j]).wait()


def kernel(input_tensor, pos_table):
    B, S, E = input_tensor.shape
    D = S * E
    x = input_tensor.reshape(B, D)
    t = pos_table.reshape(D)

    mesh = plsc.VectorSubcoreMesh(core_axis_name="c", subcore_axis_name="s",
                                  num_cores=_NC)
    scratch = (
        [pltpu.VMEM((D,), jnp.float32)]
        + [pltpu.VMEM((8, D // _NQ), jnp.float32)] * _NBUF
        + [pltpu.VMEM((8, D // _NQ), jnp.float32)] * _NBUF
        + [pltpu.SemaphoreType.DMA] * (2 * _NBUF)
    )

    def body(x_ref, t_ref, o_ref, *scr):
        t_v = scr[0]
        in_bufs = scr[1:1 + _NBUF]
        out_bufs = scr[1 + _NBUF:1 + 2 * _NBUF]
        ld_sems = scr[1 + 2 * _NBUF:1 + 3 * _NBUF]
        st_sems = scr[1 + 3 * _NBUF:1 + 4 * _NBUF]
        _sc_body(x_ref, t_ref, o_ref, t_v, in_bufs, out_bufs, ld_sems, st_sems)

    run = pl.kernel(
        body,
        out_type=jax.ShapeDtypeStruct((B, D), jnp.float32),
        mesh=mesh,
        scratch_types=scratch,
        compiler_params=pltpu.CompilerParams(use_tc_tiling_on_sc=True),
    )
    return run(x, t).reshape(B, S, E)


# speedup vs baseline: 9.5980x; 2.9708x over previous
import jax
import jax.numpy as jnp
from jax import lax
from jax.experimental import pallas as pl
from jax.experimental.pallas import tpu as pltpu
from jax.experimental.pallas import tpu_sc as plsc

_NC = 2
_NS = 16
_NW = _NC * _NS
_L = 16
_NBUF = 2
_CC = 2048


def _chunk_add(t_v, in_b, out_b, tbase):
    for s in range(8):
        tvec = t_v[pl.ds(pl.multiple_of(tbase + s * _L, _L), _L)]

        @plsc.parallel_loop(0, _CC, step=_L, unroll=8)
        def body(c):
            col = pl.multiple_of(c, _L)
            out_b[s, pl.ds(col, _L)] = in_b[s, pl.ds(col, _L)] + tvec


def _sc_body(x_hbm, t_hbm, o_hbm, t_v, in_bufs, out_bufs, ld_sems, st_sems):
    R, B = x_hbm.shape
    nq = B // _CC
    rpw = R // _NW
    trw = rpw // 8
    nch = trw * nq
    ng = nch // _NBUF
    wid = lax.axis_index("s") * _NC + lax.axis_index("c")
    wtr = wid * trw

    pltpu.sync_copy(t_hbm.at[pl.ds(wid * (rpw * _L), rpw * _L)], t_v)

    def chunk_slice(ref, e):
        tr = wtr + e // nq
        return ref.at[pl.ds(tr * 8, 8), pl.ds((e % nq) * _CC, _CC)]

    for j in range(_NBUF):
        pltpu.async_copy(chunk_slice(x_hbm, j), in_bufs[j], ld_sems[j])

    def slot(g, j, *, first, last):
        e = g * _NBUF + j
        pltpu.make_async_copy(chunk_slice(x_hbm, e), in_bufs[j], ld_sems[j]).wait()
        if not first:
            pltpu.make_async_copy(out_bufs[j], chunk_slice(o_hbm, e), st_sems[j]).wait()
        _chunk_add(t_v, in_bufs[j], out_bufs[j], (e // nq) * (8 * _L))
        pltpu.async_copy(out_bufs[j], chunk_slice(o_hbm, e), st_sems[j])
        if not last:
            pltpu.async_copy(chunk_slice(x_hbm, e + _NBUF), in_bufs[j], ld_sems[j])

    for j in range(_NBUF):
        slot(0, j, first=True, last=False)

    def group(g, c):
        for j in range(_NBUF):
            slot(g, j, first=False, last=False)
        return c

    lax.fori_loop(1, ng - 1, group, 0)

    for j in range(_NBUF):
        slot(ng - 1, j, first=False, last=True)

    for j in range(_NBUF):
        e = (ng - 1) * _NBUF + j
        pltpu.make_async_copy(out_bufs[j], chunk_slice(o_hbm, e), st_sems[j]).wait()


def kernel(input_tensor, pos_table):
    B, S, E = input_tensor.shape
    D = S * E
    x = input_tensor.transpose(1, 2, 0).reshape(D, B)
    t_rep = jnp.broadcast_to(pos_table.reshape(D, 1), (D, _L)).reshape(D * _L)

    rpw = D // _NW
    mesh = plsc.VectorSubcoreMesh(core_axis_name="c", subcore_axis_name="s",
                                  num_cores=_NC)
    scratch = (
        [pltpu.VMEM((rpw * _L,), jnp.float32)]
        + [pltpu.VMEM((8, _CC), jnp.float32)] * _NBUF
        + [pltpu.VMEM((8, _CC), jnp.float32)] * _NBUF
        + [pltpu.SemaphoreType.DMA] * (2 * _NBUF)
    )

    def body(x_ref, t_ref, o_ref, *scr):
        t_v = scr[0]
        in_bufs = scr[1:1 + _NBUF]
        out_bufs = scr[1 + _NBUF:1 + 2 * _NBUF]
        ld_sems = scr[1 + 2 * _NBUF:1 + 3 * _NBUF]
        st_sems = scr[1 + 3 * _NBUF:1 + 4 * _NBUF]
        _sc_body(x_ref, t_ref, o_ref, t_v, in_bufs, out_bufs, ld_sems, st_sems)

    run = pl.kernel(
        body,
        out_type=jax.ShapeDtypeStruct((D, B), jnp.float32),
        mesh=mesh,
        scratch_types=scratch,
        compiler_params=pltpu.CompilerParams(use_tc_tiling_on_sc=True),
    )
    out = run(x, t_rep)
    return out.reshape(S, E, B).transpose(2, 0, 1)
